# Initial kernel scaffold; baseline (speedup 1.0000x reference)
import jax
import jax.numpy as jnp
from jax import lax
from jax.experimental import pallas as pl
from jax.experimental.pallas import tpu as pltpu
from jax.experimental.pallas import tpu_sc as plsc

N = 10000
E = 320000
D = 128

NC = 2
NS = 16

ROWS_PER_TILE = N // NS
EDGES_PER_TILE = E // NS
C = 400
N_CHUNKS = EDGES_PER_TILE // C
CD = 2000
DEG_CHUNKS = EDGES_PER_TILE // CD

_MESH = plsc.VectorSubcoreMesh(
    core_axis_name="c", subcore_axis_name="s", num_cores=NC, num_subcores=NS
)


def _deg_core(dst, ones_hbm, z16, dout, s, idx_v, ones_v, acc, sem):
    row0 = s * ROWS_PER_TILE
    pltpu.sync_copy(z16.at[pl.ds(row0, ROWS_PER_TILE)],
                    acc.at[pl.ds(row0, ROWS_PER_TILE)])
    pltpu.sync_copy(ones_hbm, ones_v)
    plsc.subcore_barrier()
    ebase = s * EDGES_PER_TILE

    @pl.loop(0, DEG_CHUNKS)
    def _(i):
        b = pl.multiple_of(ebase + i * CD, 8)
        pltpu.sync_copy(dst.at[pl.ds(b, CD)], idx_v)
        pltpu.sync_copy(ones_v, acc.at[idx_v], add=True)

    plsc.subcore_barrier()
    pltpu.sync_copy(acc.at[pl.ds(row0, ROWS_PER_TILE)],
                    dout.at[pl.ds(row0, ROWS_PER_TILE)])


def _deg_body(dst0, dst1, ones_hbm, z16, d0out, d1out, idx_v, ones_v, acc, sem):
    c = lax.axis_index("c")
    s = lax.axis_index("s")

    @pl.when(c == 0)
    def _():
        _deg_core(dst0, ones_hbm, z16, d0out, s, idx_v, ones_v, acc, sem)

    @pl.when(c == 1)
    def _():
        _deg_core(dst1, ones_hbm, z16, d1out, s, idx_v, ones_v, acc, sem)


_deg_call = pl.kernel(
    _deg_body,
    out_type=(
        jax.ShapeDtypeStruct((N, 16), jnp.float32),
        jax.ShapeDtypeStruct((N, 16), jnp.float32),
    ),
    mesh=_MESH,
    scratch_types=[
        pltpu.VMEM((CD,), jnp.int32),
        pltpu.VMEM((CD, 16), jnp.float32),
        pltpu.VMEM_SHARED((N, 16), jnp.float32),
        pltpu.SemaphoreType.DMA,
    ],
)


def _scat_core(src, dst, g, z, pout, s, sidx, didx, rows, acc, sem):
    row0 = s * ROWS_PER_TILE
    pltpu.sync_copy(z.at[pl.ds(row0, ROWS_PER_TILE)],
                    acc.at[pl.ds(row0, ROWS_PER_TILE)])
    plsc.subcore_barrier()
    ebase = s * EDGES_PER_TILE

    @pl.loop(0, N_CHUNKS)
    def _(i):
        b = pl.multiple_of(ebase + i * C, 8)
        pltpu.sync_copy(src.at[pl.ds(b, C)], sidx)
        pltpu.sync_copy(dst.at[pl.ds(b, C)], didx)
        pltpu.async_copy(g.at[sidx], rows, sem).wait()
        pltpu.sync_copy(rows, acc.at[didx], add=True)

    plsc.subcore_barrier()
    pltpu.sync_copy(acc.at[pl.ds(row0, ROWS_PER_TILE)],
                    pout.at[pl.ds(row0, ROWS_PER_TILE)])


def _scat_body(src0, dst0, src1, dst1, g0, g1, z, p0, p1,
               sidx, didx, rows, acc, sem):
    c = lax.axis_index("c")
    s = lax.axis_index("s")

    @pl.when(c == 0)
    def _():
        _scat_core(src0, dst0, g0, z, p0, s, sidx, didx, rows, acc, sem)

    @pl.when(c == 1)
    def _():
        _scat_core(src1, dst1, g1, z, p1, s, sidx, didx, rows, acc, sem)


_scat_call = pl.kernel(
    _scat_body,
    out_type=(
        jax.ShapeDtypeStruct((N, D), jnp.float32),
        jax.ShapeDtypeStruct((N, D), jnp.float32),
    ),
    mesh=_MESH,
    scratch_types=[
        pltpu.VMEM((C,), jnp.int32),
        pltpu.VMEM((C,), jnp.int32),
        pltpu.VMEM((C, D), jnp.float32),
        pltpu.VMEM_SHARED((N, D), jnp.float32),
        pltpu.SemaphoreType.DMA,
    ],
)


BM = 1000


def _dinv_of(deg_blk):
    return lax.rsqrt(deg_blk[:, 0:1] + 1.0)


def _mm_g_body(x_ref, w_ref, deg_ref, h_ref, g_ref):
    x = x_ref[0]
    h = jnp.dot(x, w_ref[...], preferred_element_type=jnp.float32)
    dinv = _dinv_of(deg_ref[0])
    h_ref[0] = h
    g_ref[0] = h * dinv


def _mm_g(x, w, deg):
    return pl.pallas_call(
        _mm_g_body,
        grid=(2, N // BM),
        in_specs=[
            pl.BlockSpec((1, BM, D), lambda a, i: (a, i, 0)),
            pl.BlockSpec((D, D), lambda a, i: (0, 0)),
            pl.BlockSpec((1, BM, 16), lambda a, i: (a, i, 0)),
        ],
        out_specs=[
            pl.BlockSpec((1, BM, D), lambda a, i: (a, i, 0)),
            pl.BlockSpec((1, BM, D), lambda a, i: (a, i, 0)),
        ],
        out_shape=[
            jax.ShapeDtypeStruct((2, N, D), jnp.float32),
            jax.ShapeDtypeStruct((2, N, D), jnp.float32),
        ],
    )(x, w, deg)


def _fused_body(p_ref, hp_ref, deg_ref, b_ref, w_ref, h_ref, g_ref):
    dinv = _dinv_of(deg_ref[0])
    x = dinv * p_ref[0] + (dinv * dinv) * hp_ref[0] + b_ref[...]
    x = jnp.maximum(x, 0.0)
    h = jnp.dot(x, w_ref[...], preferred_element_type=jnp.float32)
    h_ref[0] = h
    g_ref[0] = h * dinv


def _fused(p, hp, deg, b, w):
    return pl.pallas_call(
        _fused_body,
        grid=(2, N // BM),
        in_specs=[
            pl.BlockSpec((1, BM, D), lambda a, i: (a, i, 0)),
            pl.BlockSpec((1, BM, D), lambda a, i: (a, i, 0)),
            pl.BlockSpec((1, BM, 16), lambda a, i: (a, i, 0)),
            pl.BlockSpec((1, D), lambda a, i: (0, 0)),
            pl.BlockSpec((D, D), lambda a, i: (0, 0)),
        ],
        out_specs=[
            pl.BlockSpec((1, BM, D), lambda a, i: (a, i, 0)),
            pl.BlockSpec((1, BM, D), lambda a, i: (a, i, 0)),
        ],
        out_shape=[
            jax.ShapeDtypeStruct((2, N, D), jnp.float32),
            jax.ShapeDtypeStruct((2, N, D), jnp.float32),
        ],
    )(p, hp, deg, b, w)


_LAST_BLK = N // 8 - 1


def _head_body(p_ref, hp_ref, deg_ref, b_ref, wm_ref, bm_ref, o_ref):
    dinv0 = _dinv_of(deg_ref[0])
    x0 = dinv0 * p_ref[0] + (dinv0 * dinv0) * hp_ref[0] + b_ref[...]
    dinv1 = _dinv_of(deg_ref[1])
    x1 = dinv1 * p_ref[1] + (dinv1 * dinv1) * hp_ref[1] + b_ref[...]
    l = x0[7:8, :]
    r = x1[7:8, :]
    o_ref[...] = (
        jnp.dot(l, wm_ref[0:D, :], preferred_element_type=jnp.float32)
        + jnp.dot(r, wm_ref[D:2 * D, :], preferred_element_type=jnp.float32)
        + bm_ref[...]
    )


def _head(p, hp, deg, b, wm_pad, bm_pad):
    return pl.pallas_call(
        _head_body,
        grid=(1,),
        in_specs=[
            pl.BlockSpec((2, 8, D), lambda i: (0, _LAST_BLK, 0)),
            pl.BlockSpec((2, 8, D), lambda i: (0, _LAST_BLK, 0)),
            pl.BlockSpec((2, 8, 16), lambda i: (0, _LAST_BLK, 0)),
            pl.BlockSpec((1, D), lambda i: (0, 0)),
            pl.BlockSpec((2 * D, D), lambda i: (0, 0)),
            pl.BlockSpec((1, D), lambda i: (0, 0)),
        ],
        out_specs=pl.BlockSpec((1, D), lambda i: (0, 0)),
        out_shape=jax.ShapeDtypeStruct((1, D), jnp.float32),
    )(p, hp, deg, b, wm_pad, bm_pad)


def kernel(left_x, left_edge_index, right_x, right_edge_index,
           W1, b1, W2, b2, W3, b3, Wm, bm):
    src0 = left_edge_index[0].astype(jnp.int32)
    dst0 = left_edge_index[1].astype(jnp.int32)
    src1 = right_edge_index[0].astype(jnp.int32)
    dst1 = right_edge_index[1].astype(jnp.int32)

    x = jnp.stack([left_x, right_x])

    zeros = jnp.zeros((N, D), jnp.float32)
    zeros16 = jnp.zeros((N, 16), jnp.float32)
    ones16 = jnp.ones((CD, 16), jnp.float32)

    d0, d1 = _deg_call(dst0, dst1, ones16, zeros16)
    deg = jnp.stack([d0, d1])

    b1r = b1.reshape(1, D)
    b2r = b2.reshape(1, D)
    b3r = b3.reshape(1, D)
    wm_pad = jnp.zeros((2 * D, D), jnp.float32).at[:, :2].set(Wm)
    bm_pad = jnp.zeros((1, D), jnp.float32).at[0, :2].set(bm)

    h, g = _mm_g(x, W1, deg)
    p0, p1 = _scat_call(src0, dst0, src1, dst1, g[0], g[1], zeros)
    p = jnp.stack([p0, p1])
    h, g = _fused(p, h, deg, b1r, W2)
    p0, p1 = _scat_call(src0, dst0, src1, dst1, g[0], g[1], zeros)
    p = jnp.stack([p0, p1])
    h, g = _fused(p, h, deg, b2r, W3)
    p0, p1 = _scat_call(src0, dst0, src1, dst1, g[0], g[1], zeros)
    p = jnp.stack([p0, p1])
    out = _head(p, h, deg, b3r, wm_pad, bm_pad)
    return out[:, :2]


# speedup vs baseline: 18.6083x; 18.6083x over previous
import functools

import jax
import jax.numpy as jnp
from jax import lax
from jax.experimental import pallas as pl
from jax.experimental.pallas import tpu as pltpu
from jax.experimental.pallas import tpu_sc as plsc

N = 10000
E = 320000
D = 128

NC = 2
NS = 16

ROWS_PER_TILE = 624
ROWS_TAIL = N - NS * ROWS_PER_TILE
EDGES_PER_TILE = E // NS
C = 160
N_CHUNKS = EDGES_PER_TILE // C
CD = 250
DEG_CHUNKS = EDGES_PER_TILE // CD

def _copy_tile_rows(src_ref, dst_ref, s):
    st = pl.multiple_of(s * ROWS_PER_TILE, 8)
    pltpu.sync_copy(src_ref.at[pl.ds(st, ROWS_PER_TILE)],
                    dst_ref.at[pl.ds(st, ROWS_PER_TILE)])

    @pl.when(s == NS - 1)
    def _():
        base = NS * ROWS_PER_TILE
        pltpu.sync_copy(src_ref.at[pl.ds(base, ROWS_TAIL)],
                        dst_ref.at[pl.ds(base, ROWS_TAIL)])


def _deg_core(dstR, ones_hbm, z, dout, s, di0, di1, ones_v, acc, isem):
    _copy_tile_rows(z, acc, s)
    pltpu.sync_copy(ones_hbm, ones_v)
    plsc.subcore_barrier()
    base = s * DEG_CHUNKS

    pltpu.sync_copy(dstR.at[base], di0)

    def step(i, me, nx):
        @pl.when(i + 1 < DEG_CHUNKS)
        def _():
            pltpu.async_copy(dstR.at[base + i + 1], nx, isem)
        pltpu.sync_copy(ones_v, acc.at[me], add=True)

        @pl.when(i + 1 < DEG_CHUNKS)
        def _():
            pltpu.make_async_copy(dstR.at[base + i + 1], nx, isem).wait()

    @pl.loop(0, DEG_CHUNKS)
    def _(i):
        par = lax.rem(i, 2)

        @pl.when(par == 0)
        def _():
            step(i, di0, di1)

        @pl.when(par == 1)
        def _():
            step(i, di1, di0)

    plsc.subcore_barrier()
    _copy_tile_rows(acc, dout, s)


def _deg_body(dst0, dst1, ones_hbm, z, d0out, d1out,
              di0, di1, ones_v, acc, isem):
    c = lax.axis_index("c")
    s = lax.axis_index("s")

    @pl.when(c == 0)
    def _():
        _deg_core(dst0, ones_hbm, z, d0out, s, di0, di1, ones_v, acc, isem)

    @pl.when(c == 1)
    def _():
        _deg_core(dst1, ones_hbm, z, d1out, s, di0, di1, ones_v, acc, isem)


@functools.cache
def _sc_mesh():
    return plsc.VectorSubcoreMesh(
        core_axis_name="c", subcore_axis_name="s",
        num_cores=NC, num_subcores=NS,
    )


@functools.cache
def _deg_call():
    return pl.kernel(
        _deg_body,
        out_type=(
            jax.ShapeDtypeStruct((N, D), jnp.float32),
            jax.ShapeDtypeStruct((N, D), jnp.float32),
        ),
        mesh=_sc_mesh(),
        scratch_types=[
            pltpu.VMEM((CD,), jnp.int32),
            pltpu.VMEM((CD,), jnp.int32),
            pltpu.VMEM((CD, D), jnp.float32),
            pltpu.VMEM_SHARED((N, D), jnp.float32),
            pltpu.SemaphoreType.DMA,
        ],
    )


def _scat_core(srcR, dstR, g, z, pout, s,
               si0, si1, di0, di1, rows0, rows1, acc, gsem, isem):
    _copy_tile_rows(z, acc, s)
    plsc.subcore_barrier()
    base = s * N_CHUNKS

    pltpu.sync_copy(srcR.at[base], si0)
    pltpu.sync_copy(dstR.at[base], di0)
    pltpu.async_copy(g.at[si0], rows0, gsem)

    def step(i, me_si, me_di, me_rows, nx_si, nx_di, nx_rows):
        @pl.when(i + 1 < N_CHUNKS)
        def _():
            pltpu.async_copy(srcR.at[base + i + 1], nx_si, isem)
            pltpu.async_copy(dstR.at[base + i + 1], nx_di, isem)
        pltpu.make_async_copy(g.at[me_si], me_rows, gsem).wait()

        @pl.when(i + 1 < N_CHUNKS)
        def _():
            pltpu.make_async_copy(srcR.at[base + i + 1], nx_si, isem).wait()
            pltpu.make_async_copy(dstR.at[base + i + 1], nx_di, isem).wait()
            pltpu.async_copy(g.at[nx_si], nx_rows, gsem)
        pltpu.sync_copy(me_rows, acc.at[me_di], add=True)

    @pl.loop(0, N_CHUNKS)
    def _(i):
        par = lax.rem(i, 2)

        @pl.when(par == 0)
        def _():
            step(i, si0, di0, rows0, si1, di1, rows1)

        @pl.when(par == 1)
        def _():
            step(i, si1, di1, rows1, si0, di0, rows0)

    plsc.subcore_barrier()
    _copy_tile_rows(acc, pout, s)


def _scat_body(srcR0, dstR0, srcR1, dstR1, g0, g1, z, p0, p1,
               si0, si1, di0, di1, rows0, rows1, acc, gsem, isem):
    c = lax.axis_index("c")
    s = lax.axis_index("s")

    @pl.when(c == 0)
    def _():
        _scat_core(srcR0, dstR0, g0, z, p0, s,
                   si0, si1, di0, di1, rows0, rows1, acc, gsem, isem)

    @pl.when(c == 1)
    def _():
        _scat_core(srcR1, dstR1, g1, z, p1, s,
                   si0, si1, di0, di1, rows0, rows1, acc, gsem, isem)


@functools.cache
def _scat_call():
    return pl.kernel(
        _scat_body,
        out_type=(
            jax.ShapeDtypeStruct((N, D), jnp.float32),
            jax.ShapeDtypeStruct((N, D), jnp.float32),
        ),
        mesh=_sc_mesh(),
        scratch_types=[
            pltpu.VMEM((C,), jnp.int32),
            pltpu.VMEM((C,), jnp.int32),
            pltpu.VMEM((C,), jnp.int32),
            pltpu.VMEM((C,), jnp.int32),
            pltpu.VMEM((C, D), jnp.float32),
            pltpu.VMEM((C, D), jnp.float32),
            pltpu.VMEM_SHARED((N, D), jnp.float32),
            pltpu.SemaphoreType.DMA,
            pltpu.SemaphoreType.DMA,
        ],
    )


BM = 1000


def _dinv_of(deg_blk):
    return lax.rsqrt(deg_blk[:, 0:1] + 1.0)


def _mm_g_body(x_ref, w_ref, deg_ref, h_ref, g_ref):
    x = x_ref[0]
    h = jnp.dot(x, w_ref[...], preferred_element_type=jnp.float32)
    dinv = _dinv_of(deg_ref[0])
    h_ref[0] = h
    g_ref[0] = h * dinv


def _mm_g(x, w, deg):
    return pl.pallas_call(
        _mm_g_body,
        grid=(2, N // BM),
        in_specs=[
            pl.BlockSpec((1, BM, D), lambda a, i: (a, i, 0)),
            pl.BlockSpec((D, D), lambda a, i: (0, 0)),
            pl.BlockSpec((1, BM, D), lambda a, i: (a, i, 0)),
        ],
        out_specs=[
            pl.BlockSpec((1, BM, D), lambda a, i: (a, i, 0)),
            pl.BlockSpec((1, BM, D), lambda a, i: (a, i, 0)),
        ],
        out_shape=[
            jax.ShapeDtypeStruct((2, N, D), jnp.float32),
            jax.ShapeDtypeStruct((2, N, D), jnp.float32),
        ],
    )(x, w, deg)


def _fused_body(p_ref, hp_ref, deg_ref, b_ref, w_ref, h_ref, g_ref):
    dinv = _dinv_of(deg_ref[0])
    x = dinv * p_ref[0] + (dinv * dinv) * hp_ref[0] + b_ref[...]
    x = jnp.maximum(x, 0.0)
    h = jnp.dot(x, w_ref[...], preferred_element_type=jnp.float32)
    h_ref[0] = h
    g_ref[0] = h * dinv


def _fused(p, hp, deg, b, w):
    return pl.pallas_call(
        _fused_body,
        grid=(2, N // BM),
        in_specs=[
            pl.BlockSpec((1, BM, D), lambda a, i: (a, i, 0)),
            pl.BlockSpec((1, BM, D), lambda a, i: (a, i, 0)),
            pl.BlockSpec((1, BM, D), lambda a, i: (a, i, 0)),
            pl.BlockSpec((1, D), lambda a, i: (0, 0)),
            pl.BlockSpec((D, D), lambda a, i: (0, 0)),
        ],
        out_specs=[
            pl.BlockSpec((1, BM, D), lambda a, i: (a, i, 0)),
            pl.BlockSpec((1, BM, D), lambda a, i: (a, i, 0)),
        ],
        out_shape=[
            jax.ShapeDtypeStruct((2, N, D), jnp.float32),
            jax.ShapeDtypeStruct((2, N, D), jnp.float32),
        ],
    )(p, hp, deg, b, w)


_LAST_BLK = N // 8 - 1


def _head_body(p_ref, hp_ref, deg_ref, b_ref, wm_ref, bm_ref, o_ref):
    dinv0 = _dinv_of(deg_ref[0])[7:8]
    dinv1 = _dinv_of(deg_ref[1])[7:8]
    p0 = p_ref[0, 7:8, :]
    p1 = p_ref[1, 7:8, :]
    l = dinv0 * p0 + (dinv0 * dinv0) * hp_ref[0, 7:8, :] + b_ref[...]
    r = dinv1 * p1 + (dinv1 * dinv1) * hp_ref[1, 7:8, :] + b_ref[...]
    o_ref[...] = (
        jnp.dot(l, wm_ref[0:D, :], preferred_element_type=jnp.float32)
        + jnp.dot(r, wm_ref[D:2 * D, :], preferred_element_type=jnp.float32)
        + bm_ref[...]
    )


def _head(p, hp, deg, b, wm_pad, bm_pad):
    return pl.pallas_call(
        _head_body,
        grid=(1,),
        in_specs=[
            pl.BlockSpec((2, 8, D), lambda i: (0, _LAST_BLK, 0)),
            pl.BlockSpec((2, 8, D), lambda i: (0, _LAST_BLK, 0)),
            pl.BlockSpec((2, 8, D), lambda i: (0, _LAST_BLK, 0)),
            pl.BlockSpec((1, D), lambda i: (0, 0)),
            pl.BlockSpec((2 * D, D), lambda i: (0, 0)),
            pl.BlockSpec((1, D), lambda i: (0, 0)),
        ],
        out_specs=pl.BlockSpec((1, D), lambda i: (0, 0)),
        out_shape=jax.ShapeDtypeStruct((1, D), jnp.float32),
    )(p, hp, deg, b, wm_pad, bm_pad)


def kernel(left_x, left_edge_index, right_x, right_edge_index,
           W1, b1, W2, b2, W3, b3, Wm, bm):
    src0 = left_edge_index[0].astype(jnp.int32)
    dst0 = left_edge_index[1].astype(jnp.int32)
    src1 = right_edge_index[0].astype(jnp.int32)
    dst1 = right_edge_index[1].astype(jnp.int32)

    srcC0 = src0.reshape(-1, C)
    dstC0 = dst0.reshape(-1, C)
    srcC1 = src1.reshape(-1, C)
    dstC1 = dst1.reshape(-1, C)
    dstR0 = dst0.reshape(-1, CD)
    dstR1 = dst1.reshape(-1, CD)

    x = jnp.stack([left_x, right_x])

    zeros = jnp.zeros((N, D), jnp.float32)
    ones_rows = jnp.ones((CD, D), jnp.float32)

    d0, d1 = _deg_call()(dstR0, dstR1, ones_rows, zeros)
    deg = jnp.stack([d0, d1])

    b1r = b1.reshape(1, D)
    b2r = b2.reshape(1, D)
    b3r = b3.reshape(1, D)
    wm_pad = jnp.zeros((2 * D, D), jnp.float32).at[:, :2].set(Wm)
    bm_pad = jnp.zeros((1, D), jnp.float32).at[0, :2].set(bm)

    h, g = _mm_g(x, W1, deg)
    p0, p1 = _scat_call()(srcC0, dstC0, srcC1, dstC1, g[0], g[1], zeros)
    p = jnp.stack([p0, p1])
    h, g = _fused(p, h, deg, b1r, W2)
    p0, p1 = _scat_call()(srcC0, dstC0, srcC1, dstC1, g[0], g[1], zeros)
    p = jnp.stack([p0, p1])
    h, g = _fused(p, h, deg, b2r, W3)
    p0, p1 = _scat_call()(srcC0, dstC0, srcC1, dstC1, g[0], g[1], zeros)
    p = jnp.stack([p0, p1])
    out = _head(p, h, deg, b3r, wm_pad, bm_pad)
    return out[:, :2]
